# baseline (speedup 1.0000x reference)
import jax
import jax.numpy as jnp
from jax import lax
from jax.experimental import pallas as pl
from jax.experimental.pallas import tpu as pltpu
from jax.experimental.pallas import tpu_sc as plsc

_VOCAB = 1000000
_EMB = 64
_HID = 64
_PADW = 128
_B = 4096
_L = 200


_MM_BLK = 2000


def _mm_body(t_ref, w_ref, o_ref):
    o_ref[:, :_HID] = lax.dot_general(
        t_ref[...], w_ref[...], (((1,), (1,)), ((), ())),
        preferred_element_type=jnp.float32)


def _project(table, W):
    return pl.pallas_call(
        _mm_body,
        grid=(_VOCAB // _MM_BLK,),
        in_specs=[
            pl.BlockSpec((_MM_BLK, _EMB), lambda i: (i, 0)),
            pl.BlockSpec((_HID, _EMB), lambda i: (0, 0)),
        ],
        out_specs=pl.BlockSpec((_MM_BLK, _PADW), lambda i: (i, 0)),
        out_shape=jax.ShapeDtypeStruct((_VOCAB, _PADW), jnp.float32),
    )(table, W)


_NC, _NS = 2, 16
_NW = _NC * _NS
_BATCH_PER_W = _B // _NW
_WB = 2
_HALF_WAVES = _BATCH_PER_W // _WB // 2
_SPLITS = ((0, 128), (128, 72))


def _fire(proj, idx_hbm, b0, idx_v, rows_v, sem):
    pltpu.sync_copy(idx_hbm.at[pl.ds(b0, _WB)], idx_v)
    return [
        pltpu.async_copy(
            proj.at[idx_v.at[i, pl.ds(off, ln)]],
            rows_v.at[i, pl.ds(off, ln)], sem)
        for i in range(_WB) for off, ln in _SPLITS
    ]


def _drain_store(out_hbm, b0, rows_v, copies):
    for c in copies:
        c.wait()
    pltpu.sync_copy(rows_v, out_hbm.at[pl.ds(b0, _WB)])


def _gather_body(proj, idx_hbm, out_hbm,
                 idx_a, idx_b, rows_a, rows_b, sem_a, sem_b):
    wid = lax.axis_index("s") * _NC + lax.axis_index("c")
    bbase = wid * _BATCH_PER_W

    def body(w, carry):
        b_a = bbase + (2 * w) * _WB
        b_b = bbase + (2 * w + 1) * _WB
        cps_a = _fire(proj, idx_hbm, b_a, idx_a, rows_a, sem_a)
        cps_b = _fire(proj, idx_hbm, b_b, idx_b, rows_b, sem_b)
        _drain_store(out_hbm, b_a, rows_a, cps_a)
        _drain_store(out_hbm, b_b, rows_b, cps_b)
        return carry
    lax.fori_loop(0, _HALF_WAVES, body, 0)


_gather = pl.kernel(
    _gather_body,
    out_type=jax.ShapeDtypeStruct((_B, _L, _PADW), jnp.float32),
    mesh=plsc.VectorSubcoreMesh(core_axis_name="c", subcore_axis_name="s"),
    scratch_types=[
        pltpu.VMEM((_WB, _L), jnp.int32),
        pltpu.VMEM((_WB, _L), jnp.int32),
        pltpu.VMEM((_WB, _L, _PADW), jnp.float32),
        pltpu.VMEM((_WB, _L, _PADW), jnp.float32),
        pltpu.SemaphoreType.DMA,
        pltpu.SemaphoreType.DMA,
    ],
    compiler_params=pltpu.CompilerParams(use_tc_tiling_on_sc=False),
)


_SB = 64


def _slice_body(g_ref, o_ref):
    o_ref[...] = g_ref[:, :, :_HID]


def _slice64(g):
    return pl.pallas_call(
        _slice_body,
        grid=(_B // _SB,),
        in_specs=[pl.BlockSpec((_SB, _L, _PADW), lambda i: (i, 0, 0))],
        out_specs=pl.BlockSpec((_SB, _L, _HID), lambda i: (i, 0, 0)),
        out_shape=jax.ShapeDtypeStruct((_B, _L, _HID), jnp.float32),
    )(g)


def kernel(prem, hypo, table, W):
    proj = _project(table, W)
    gp = _gather(proj, prem.astype(jnp.int32))
    gh = _gather(proj, hypo.astype(jnp.int32))
    return _slice64(gp), _slice64(gh)


# speedup vs baseline: 1.7190x; 1.7190x over previous
import jax
import jax.numpy as jnp
from jax import lax
from jax.experimental import pallas as pl
from jax.experimental.pallas import tpu as pltpu
from jax.experimental.pallas import tpu_sc as plsc

_VOCAB = 1000000
_EMB = 64
_HID = 64
_PADW = 128
_B = 4096
_L = 200


_MM_BLK = 2048


def _mm_body(tT_ref, w_ref, o_ref):
    y = lax.dot_general(
        w_ref[...], tT_ref[...], (((1,), (0,)), ((), ())),
        preferred_element_type=jnp.float32)
    o_ref[:, :_HID] = y.T


def _project(tableT, W):
    return pl.pallas_call(
        _mm_body,
        grid=(pl.cdiv(_VOCAB, _MM_BLK),),
        in_specs=[
            pl.BlockSpec((_EMB, _MM_BLK), lambda i: (0, i)),
            pl.BlockSpec((_HID, _EMB), lambda i: (0, 0)),
        ],
        out_specs=pl.BlockSpec((_MM_BLK, _PADW), lambda i: (i, 0)),
        out_shape=jax.ShapeDtypeStruct((_VOCAB, _PADW), jnp.float32),
    )(tableT, W)


_NC, _NS = 2, 16
_NW = _NC * _NS
_BATCH_PER_W = _B // _NW
_WB = 2
_HALF_WAVES = _BATCH_PER_W // _WB // 2
_SPLITS = ((0, 128), (128, 72))


def _fire(proj, idx_hbm, b0, idx_v, rows_v, sem):
    pltpu.sync_copy(idx_hbm.at[pl.ds(b0, _WB)], idx_v)
    return [
        pltpu.async_copy(
            proj.at[idx_v.at[i, pl.ds(off, ln)]],
            rows_v.at[i, pl.ds(off, ln)], sem)
        for i in range(_WB) for off, ln in _SPLITS
    ]


def _drain_store(out_hbm, b0, rows_v, copies):
    for c in copies:
        c.wait()
    pltpu.sync_copy(rows_v, out_hbm.at[pl.ds(b0, _WB)])


def _gather_body(proj, idx_hbm, out_hbm,
                 idx_a, idx_b, rows_a, rows_b, sem_a, sem_b):
    wid = lax.axis_index("s") * _NC + lax.axis_index("c")
    bbase = wid * _BATCH_PER_W

    def body(w, carry):
        b_a = bbase + (2 * w) * _WB
        b_b = bbase + (2 * w + 1) * _WB
        cps_a = _fire(proj, idx_hbm, b_a, idx_a, rows_a, sem_a)
        cps_b = _fire(proj, idx_hbm, b_b, idx_b, rows_b, sem_b)
        _drain_store(out_hbm, b_a, rows_a, cps_a)
        _drain_store(out_hbm, b_b, rows_b, cps_b)
        return carry
    lax.fori_loop(0, _HALF_WAVES, body, 0)


_gather = pl.kernel(
    _gather_body,
    out_type=jax.ShapeDtypeStruct((_B, _L, _PADW), jnp.float32),
    mesh=plsc.VectorSubcoreMesh(core_axis_name="c", subcore_axis_name="s"),
    scratch_types=[
        pltpu.VMEM((_WB, _L), jnp.int32),
        pltpu.VMEM((_WB, _L), jnp.int32),
        pltpu.VMEM((_WB, _L, _PADW), jnp.float32),
        pltpu.VMEM((_WB, _L, _PADW), jnp.float32),
        pltpu.SemaphoreType.DMA,
        pltpu.SemaphoreType.DMA,
    ],
    compiler_params=pltpu.CompilerParams(use_tc_tiling_on_sc=False),
)


def kernel(prem, hypo, table, W):
    proj = _project(table.T, W)
    gp = _gather(proj, prem.astype(jnp.int32))
    gh = _gather(proj, hypo.astype(jnp.int32))
    return gp[:, :, :_HID], gh[:, :, :_HID]


# speedup vs baseline: 1.7426x; 1.0137x over previous
import jax
import jax.numpy as jnp
from jax import lax
from jax.experimental import pallas as pl
from jax.experimental.pallas import tpu as pltpu
from jax.experimental.pallas import tpu_sc as plsc

_VOCAB = 1000000
_EMB = 64
_HID = 64
_PADW = 128
_B = 4096
_L = 200


_MM_BLK = 2048


def _mm_body(tT_ref, w_ref, o_ref):
    y = lax.dot_general(
        w_ref[...], tT_ref[...], (((1,), (0,)), ((), ())),
        preferred_element_type=jnp.float32)
    o_ref[:, :_HID] = y.T


def _project(tableT, W):
    return pl.pallas_call(
        _mm_body,
        grid=(pl.cdiv(_VOCAB, _MM_BLK),),
        in_specs=[
            pl.BlockSpec((_EMB, _MM_BLK), lambda i: (0, i)),
            pl.BlockSpec((_HID, _EMB), lambda i: (0, 0)),
        ],
        out_specs=pl.BlockSpec((_MM_BLK, _PADW), lambda i: (i, 0)),
        out_shape=jax.ShapeDtypeStruct((_VOCAB, _PADW), jnp.float32),
    )(tableT, W)


_NC, _NS = 2, 16
_NW = _NC * _NS
_BATCH_PER_W = _B // _NW
_NBUF = 4
_RWAVES = _BATCH_PER_W // _NBUF
_SPLITS = ((0, 128), (128, 72))


def _fire(proj, idx_hbm, b, idx_v, rows_v, sem):
    pltpu.sync_copy(idx_hbm.at[b], idx_v)
    return [
        pltpu.async_copy(
            proj.at[idx_v.at[pl.ds(off, ln)]],
            rows_v.at[pl.ds(off, ln)], sem)
        for off, ln in _SPLITS
    ]


def _drain_store(out_hbm, b, rows_v, copies):
    for c in copies:
        c.wait()
    pltpu.sync_copy(rows_v, out_hbm.at[b])


def _gather_body(proj, idx_hbm, out_hbm, idx_v, rows_v, sems):
    wid = lax.axis_index("s") * _NC + lax.axis_index("c")
    bbase = wid * _BATCH_PER_W

    def body(w, carry):
        b0 = bbase + w * _NBUF
        cps = [
            _fire(proj, idx_hbm, b0 + k, idx_v.at[k], rows_v.at[k],
                  sems.at[k])
            for k in range(_NBUF)
        ]
        for k in range(_NBUF):
            _drain_store(out_hbm, b0 + k, rows_v.at[k], cps[k])
        return carry
    lax.fori_loop(0, _RWAVES, body, 0)


_gather = pl.kernel(
    _gather_body,
    out_type=jax.ShapeDtypeStruct((_B, _L, _PADW), jnp.float32),
    mesh=plsc.VectorSubcoreMesh(core_axis_name="c", subcore_axis_name="s"),
    scratch_types=[
        pltpu.VMEM((_NBUF, _L), jnp.int32),
        pltpu.VMEM((_NBUF, _L, _PADW), jnp.float32),
        pltpu.SemaphoreType.DMA((_NBUF,)),
    ],
    compiler_params=pltpu.CompilerParams(use_tc_tiling_on_sc=False),
)


def kernel(prem, hypo, table, W):
    proj = _project(table.T, W)
    gp = _gather(proj, prem.astype(jnp.int32))
    gh = _gather(proj, hypo.astype(jnp.int32))
    return gp[:, :, :_HID], gh[:, :, :_HID]
